# baseline (speedup 1.0000x reference)
import jax
import jax.numpy as jnp
from jax.experimental import pallas as pl
from jax.experimental.pallas import tpu as pltpu

_BT = 64


def _lenet_kernel(xr_ref, wb1_ref, wb2_ref, g_ref,
                  fc1b_ref, fc2w_ref, fc2b_ref, fc3w_ref, fc3b_ref,
                  out_ref):
    f32, bf16 = jnp.float32, jnp.bfloat16
    bt = xr_ref.shape[0]

    def shift_rows(x, k):
        return jnp.concatenate([x[:, k:, :], x[:, :k, :]], axis=1)

    def shift_lanes(x, k):
        return jnp.concatenate([x[:, :, k:], x[:, :, :k]], axis=2)

    x2 = xr_ref[...]

    xbig = jnp.concatenate(
        [x2] + [shift_rows(x2, i) for i in range(1, 5)], axis=2)
    acc = jnp.dot(xbig.reshape(bt * 32, 640), wb1_ref[...],
                  preferred_element_type=f32).reshape(bt, 32, 256)

    u = jnp.maximum(acc, shift_rows(acc, 1))
    p1 = jnp.maximum(jnp.maximum(u, shift_lanes(u, 8)), 0.0).astype(bf16)

    pbig = jnp.concatenate(
        [p1] + [shift_rows(p1, 2 * i) for i in range(1, 5)], axis=2)
    acc2 = jnp.dot(pbig.reshape(bt * 32, 1280), wb2_ref[...],
                   preferred_element_type=f32).reshape(bt, 32, 160)

    u2 = jnp.maximum(acc2, shift_rows(acc2, 2))
    p2 = jnp.maximum(jnp.maximum(u2, shift_lanes(u2, 16)), 0.0).astype(bf16)

    h = jnp.dot(p2[:, 0, :], g_ref[0], preferred_element_type=f32)
    for k in range(1, 5):
        h = h + jnp.dot(p2[:, 4 * k, :], g_ref[k], preferred_element_type=f32)
    h = jnp.maximum(h + fc1b_ref[...], 0.0)
    h = jnp.dot(h.astype(bf16), fc2w_ref[...], preferred_element_type=f32)
    h = jnp.maximum(h + fc2b_ref[...], 0.0)
    h = jnp.dot(h.astype(bf16), fc3w_ref[...], preferred_element_type=f32)
    out_ref[...] = h + fc3b_ref[...]


def _forward(w1, b1, w2, b2, s2, fc1_w, fc1_b, fc2_w, fc2_b, fc3_w, fc3_b, x):
    del s2
    f32, bf16 = jnp.float32, jnp.bfloat16
    b = x.shape[0]
    bt = _BT
    bp = ((b + bt - 1) // bt) * bt
    nb = bp // bt

    xt = jnp.transpose(x.astype(bf16), (0, 2, 1, 3)).reshape(b, 32, 96)
    xin = jnp.concatenate(
        [xt, jnp.ones((b, 32, 1), bf16), jnp.zeros((b, 32, 31), bf16)],
        axis=2)
    if bp != b:
        xin = jnp.pad(xin, ((0, bp - b), (0, 0), (0, 0)))

    w1r = w1.astype(f32).reshape(5, 5, 3, 8)
    band1 = (jnp.arange(32)[None, :, None]
             == jnp.arange(28)[None, None, :]
             + jnp.arange(5)[:, None, None]).astype(f32)
    wb1 = jnp.einsum('ijco,jwp->icwpo', w1r, band1)
    wb1 = wb1.reshape(5, 3, 32, 224)
    wb1 = jnp.pad(wb1, ((0, 0), (0, 1), (0, 0), (0, 32)))
    brow1 = jnp.concatenate(
        [jnp.tile(b1[0], 28), jnp.ones((1,), f32), jnp.zeros((31,), f32)])
    wb1 = wb1.at[0, 3, 0, :].set(brow1)
    wb1 = wb1.reshape(640, 256).astype(bf16)

    w2f = w2.astype(f32).reshape(5, 5, 8, 16)
    w2f = jnp.pad(w2f, ((0, 0), (0, 0), (0, 8), (0, 0)))
    band2 = (jnp.arange(14)[None, :, None]
             == jnp.arange(10)[None, None, :]
             + jnp.arange(5)[:, None, None]).astype(f32)
    wb2 = jnp.einsum('ijco,jwp->iwcpo', w2f, band2)
    wb2 = wb2.reshape(5, 224, 160)
    wb2 = jnp.pad(wb2, ((0, 0), (0, 32), (0, 0)))
    wb2 = wb2.at[0, 224, :].set(jnp.tile(b2[0], 10))
    wb2 = wb2.reshape(1280, 160).astype(bf16)

    g = fc1_w[:400].reshape(5, 5, 16, 128)
    g = jnp.pad(g, ((0, 0), (0, 0), (0, 16), (0, 0))).reshape(5, 160, 128)

    c2 = lambda i: (0, 0)
    c3m = lambda i: (0, 0, 0)
    out = pl.pallas_call(
        _lenet_kernel,
        out_shape=jax.ShapeDtypeStruct((bp, 128), f32),
        grid=(nb,),
        in_specs=[
            pl.BlockSpec((bt, 32, 128), lambda i: (i, 0, 0)),
            pl.BlockSpec((640, 256), c2),
            pl.BlockSpec((1280, 160), c2),
            pl.BlockSpec((5, 160, 128), c3m),
            pl.BlockSpec((1, 128), c2),
            pl.BlockSpec((128, 128), c2),
            pl.BlockSpec((1, 128), c2),
            pl.BlockSpec((128, 128), c2),
            pl.BlockSpec((1, 128), c2),
        ],
        out_specs=pl.BlockSpec((bt, 128), lambda i: (i, 0)),
        compiler_params=pltpu.CompilerParams(
            dimension_semantics=("parallel",),
            vmem_limit_bytes=64 * 1024 * 1024),
    )(xin, wb1, wb2, g, fc1_b, fc2_w, fc2_b, fc3_w, fc3_b)
    return out[:b, :10]


_forward_jit = jax.jit(_forward)


def kernel(w1, b1, w2, b2, s2, fc1_w, fc1_b, fc2_w, fc2_b, fc3_w, fc3_b, x):
    return _forward_jit(w1, b1, w2, b2, s2, fc1_w, fc1_b, fc2_w, fc2_b,
                        fc3_w, fc3_b, x)


# speedup vs baseline: 1.0644x; 1.0644x over previous
import jax
import jax.numpy as jnp
from jax.experimental import pallas as pl
from jax.experimental.pallas import tpu as pltpu

_BT = 128


def _lenet_kernel(xr_ref, wb1_ref, wb2_ref, g_ref,
                  fc1b_ref, fc2w_ref, fc2b_ref, fc3w_ref, fc3b_ref,
                  out_ref):
    f32, bf16 = jnp.float32, jnp.bfloat16
    bt = xr_ref.shape[0]

    def shift_rows(x, k):
        return jnp.concatenate([x[:, k:, :], x[:, :k, :]], axis=1)

    def shift_lanes(x, k):
        return jnp.concatenate([x[:, :, k:], x[:, :, :k]], axis=2)

    xr = xr_ref[...]
    x2 = jnp.concatenate(
        [xr, jnp.ones((bt, 32, 1), bf16), jnp.zeros((bt, 32, 31), bf16)],
        axis=2)

    xbig = jnp.concatenate(
        [x2] + [shift_rows(x2, i) for i in range(1, 5)], axis=2)
    acc = jnp.dot(xbig.reshape(bt * 32, 640), wb1_ref[...],
                  preferred_element_type=f32)
    o1 = jnp.maximum(acc, 0.0).reshape(bt, 32, 256)

    u = jnp.maximum(o1, shift_rows(o1, 1))
    p1 = jnp.maximum(u, shift_lanes(u, 8)).astype(bf16)

    pbig = jnp.concatenate(
        [p1] + [shift_rows(p1, 2 * i) for i in range(1, 5)], axis=2)
    acc2 = jnp.dot(pbig.reshape(bt * 32, 1280), wb2_ref[...],
                   preferred_element_type=f32)
    o2 = jnp.maximum(acc2, 0.0).reshape(bt, 32, 160)

    u2 = jnp.maximum(o2, shift_rows(o2, 2))
    p2 = jnp.maximum(u2, shift_lanes(u2, 16)).astype(bf16)

    h = jnp.dot(p2[:, 0, :], g_ref[0], preferred_element_type=f32)
    for k in range(1, 5):
        h = h + jnp.dot(p2[:, 4 * k, :], g_ref[k], preferred_element_type=f32)
    h = jnp.maximum(h + fc1b_ref[...], 0.0)
    h = jnp.dot(h.astype(bf16), fc2w_ref[...], preferred_element_type=f32)
    h = jnp.maximum(h + fc2b_ref[...], 0.0)
    h = jnp.dot(h.astype(bf16), fc3w_ref[...], preferred_element_type=f32)
    out_ref[...] = h + fc3b_ref[...]


def _forward(w1, b1, w2, b2, s2, fc1_w, fc1_b, fc2_w, fc2_b, fc3_w, fc3_b, x):
    del s2
    f32, bf16 = jnp.float32, jnp.bfloat16
    b = x.shape[0]
    bt = _BT
    bp = ((b + bt - 1) // bt) * bt
    nb = bp // bt

    xin = jnp.transpose(x.astype(bf16), (0, 2, 1, 3)).reshape(b, 32, 96)
    if bp != b:
        xin = jnp.pad(xin, ((0, bp - b), (0, 0), (0, 0)))

    w1r = w1.astype(f32).reshape(5, 5, 3, 8)
    band1 = (jnp.arange(32)[None, :, None]
             == jnp.arange(28)[None, None, :]
             + jnp.arange(5)[:, None, None]).astype(f32)
    wb1 = jnp.einsum('ijco,jwp->icwpo', w1r, band1)
    wb1 = wb1.reshape(5, 3, 32, 224)
    wb1 = jnp.pad(wb1, ((0, 0), (0, 1), (0, 0), (0, 32)))
    brow1 = jnp.concatenate(
        [jnp.tile(b1[0], 28), jnp.ones((1,), f32), jnp.zeros((31,), f32)])
    wb1 = wb1.at[0, 3, 0, :].set(brow1)
    wb1 = wb1.reshape(640, 256).astype(bf16)

    w2f = w2.astype(f32).reshape(5, 5, 8, 16)
    w2f = jnp.pad(w2f, ((0, 0), (0, 0), (0, 8), (0, 0)))
    band2 = (jnp.arange(14)[None, :, None]
             == jnp.arange(10)[None, None, :]
             + jnp.arange(5)[:, None, None]).astype(f32)
    wb2 = jnp.einsum('ijco,jwp->iwcpo', w2f, band2)
    wb2 = wb2.reshape(5, 224, 160)
    wb2 = jnp.pad(wb2, ((0, 0), (0, 32), (0, 0)))
    wb2 = wb2.at[0, 224, :].set(jnp.tile(b2[0], 10))
    wb2 = wb2.reshape(1280, 160).astype(bf16)

    g = fc1_w[:400].reshape(5, 5, 16, 128)
    g = jnp.pad(g, ((0, 0), (0, 0), (0, 16), (0, 0))).reshape(5, 160, 128)

    c2 = lambda i: (0, 0)
    c3m = lambda i: (0, 0, 0)
    out = pl.pallas_call(
        _lenet_kernel,
        out_shape=jax.ShapeDtypeStruct((bp, 128), f32),
        grid=(nb,),
        in_specs=[
            pl.BlockSpec((bt, 32, 96), lambda i: (i, 0, 0)),
            pl.BlockSpec((640, 256), c2),
            pl.BlockSpec((1280, 160), c2),
            pl.BlockSpec((5, 160, 128), c3m),
            pl.BlockSpec((1, 128), c2),
            pl.BlockSpec((128, 128), c2),
            pl.BlockSpec((1, 128), c2),
            pl.BlockSpec((128, 128), c2),
            pl.BlockSpec((1, 128), c2),
        ],
        out_specs=pl.BlockSpec((bt, 128), lambda i: (i, 0)),
        compiler_params=pltpu.CompilerParams(
            dimension_semantics=("parallel",),
            vmem_limit_bytes=64 * 1024 * 1024),
    )(xin, wb1, wb2, g, fc1_b, fc2_w, fc2_b, fc3_w, fc3_b)
    return out[:b, :10]


_forward_jit = jax.jit(_forward)


def kernel(w1, b1, w2, b2, s2, fc1_w, fc1_b, fc2_w, fc2_b, fc3_w, fc3_b, x):
    return _forward_jit(w1, b1, w2, b2, s2, fc1_w, fc1_b, fc2_w, fc2_b,
                        fc3_w, fc3_b, x)


# speedup vs baseline: 1.0858x; 1.0201x over previous
import jax
import jax.numpy as jnp
from jax.experimental import pallas as pl
from jax.experimental.pallas import tpu as pltpu

_BT = 256


def _lenet_kernel(xr_ref, wb1_ref, wb2_ref, g_ref,
                  fc1b_ref, fc2w_ref, fc2b_ref, fc3w_ref, fc3b_ref,
                  out_ref):
    f32, bf16 = jnp.float32, jnp.bfloat16
    bt = xr_ref.shape[0]

    def shift_rows(x, k):
        return jnp.concatenate([x[:, k:, :], x[:, :k, :]], axis=1)

    def shift_lanes(x, k):
        return jnp.concatenate([x[:, :, k:], x[:, :, :k]], axis=2)

    xr = xr_ref[...]
    x2 = jnp.concatenate(
        [xr, jnp.ones((bt, 32, 1), bf16), jnp.zeros((bt, 32, 31), bf16)],
        axis=2)

    xbig = jnp.concatenate(
        [x2] + [shift_rows(x2, i) for i in range(1, 5)], axis=2)
    acc = jnp.dot(xbig.reshape(bt * 32, 640), wb1_ref[...],
                  preferred_element_type=f32)
    o1 = jnp.maximum(acc, 0.0).reshape(bt, 32, 256)

    u = jnp.maximum(o1, shift_rows(o1, 1))
    p1 = jnp.maximum(u, shift_lanes(u, 8)).astype(bf16)

    pbig = jnp.concatenate(
        [p1] + [shift_rows(p1, 2 * i) for i in range(1, 5)], axis=2)
    acc2 = jnp.dot(pbig.reshape(bt * 32, 1280), wb2_ref[...],
                   preferred_element_type=f32)
    o2 = jnp.maximum(acc2, 0.0).reshape(bt, 32, 160)

    u2 = jnp.maximum(o2, shift_rows(o2, 2))
    p2 = jnp.maximum(u2, shift_lanes(u2, 16)).astype(bf16)

    h = jnp.dot(p2[:, 0, :], g_ref[0], preferred_element_type=f32)
    for k in range(1, 5):
        h = h + jnp.dot(p2[:, 4 * k, :], g_ref[k], preferred_element_type=f32)
    h = jnp.maximum(h + fc1b_ref[...], 0.0)
    h = jnp.dot(h.astype(bf16), fc2w_ref[...], preferred_element_type=f32)
    h = jnp.maximum(h + fc2b_ref[...], 0.0)
    h = jnp.dot(h.astype(bf16), fc3w_ref[...], preferred_element_type=f32)
    out_ref[...] = h + fc3b_ref[...]


def _forward(w1, b1, w2, b2, s2, fc1_w, fc1_b, fc2_w, fc2_b, fc3_w, fc3_b, x):
    del s2
    f32, bf16 = jnp.float32, jnp.bfloat16
    b = x.shape[0]
    bt = _BT
    bp = ((b + bt - 1) // bt) * bt
    nb = bp // bt

    xin = jnp.transpose(x.astype(bf16), (0, 2, 1, 3)).reshape(b, 32, 96)
    if bp != b:
        xin = jnp.pad(xin, ((0, bp - b), (0, 0), (0, 0)))

    w1r = w1.astype(f32).reshape(5, 5, 3, 8)
    band1 = (jnp.arange(32)[None, :, None]
             == jnp.arange(28)[None, None, :]
             + jnp.arange(5)[:, None, None]).astype(f32)
    wb1 = jnp.einsum('ijco,jwp->icwpo', w1r, band1)
    wb1 = wb1.reshape(5, 3, 32, 224)
    wb1 = jnp.pad(wb1, ((0, 0), (0, 1), (0, 0), (0, 32)))
    brow1 = jnp.concatenate(
        [jnp.tile(b1[0], 28), jnp.ones((1,), f32), jnp.zeros((31,), f32)])
    wb1 = wb1.at[0, 3, 0, :].set(brow1)
    wb1 = wb1.reshape(640, 256).astype(bf16)

    w2f = w2.astype(f32).reshape(5, 5, 8, 16)
    w2f = jnp.pad(w2f, ((0, 0), (0, 0), (0, 8), (0, 0)))
    band2 = (jnp.arange(14)[None, :, None]
             == jnp.arange(10)[None, None, :]
             + jnp.arange(5)[:, None, None]).astype(f32)
    wb2 = jnp.einsum('ijco,jwp->iwcpo', w2f, band2)
    wb2 = wb2.reshape(5, 224, 160)
    wb2 = jnp.pad(wb2, ((0, 0), (0, 32), (0, 0)))
    wb2 = wb2.at[0, 224, :].set(jnp.tile(b2[0], 10))
    wb2 = wb2.reshape(1280, 160).astype(bf16)

    g = fc1_w[:400].reshape(5, 5, 16, 128)
    g = jnp.pad(g, ((0, 0), (0, 0), (0, 16), (0, 0))).reshape(5, 160, 128)

    c2 = lambda i: (0, 0)
    c3m = lambda i: (0, 0, 0)
    out = pl.pallas_call(
        _lenet_kernel,
        out_shape=jax.ShapeDtypeStruct((bp, 128), f32),
        grid=(nb,),
        in_specs=[
            pl.BlockSpec((bt, 32, 96), lambda i: (i, 0, 0)),
            pl.BlockSpec((640, 256), c2),
            pl.BlockSpec((1280, 160), c2),
            pl.BlockSpec((5, 160, 128), c3m),
            pl.BlockSpec((1, 128), c2),
            pl.BlockSpec((128, 128), c2),
            pl.BlockSpec((1, 128), c2),
            pl.BlockSpec((128, 128), c2),
            pl.BlockSpec((1, 128), c2),
        ],
        out_specs=pl.BlockSpec((bt, 128), lambda i: (i, 0)),
        compiler_params=pltpu.CompilerParams(
            dimension_semantics=("parallel",),
            vmem_limit_bytes=64 * 1024 * 1024),
    )(xin, wb1, wb2, g, fc1_b, fc2_w, fc2_b, fc3_w, fc3_b)
    return out[:b, :10]


_forward_jit = jax.jit(_forward)


def kernel(w1, b1, w2, b2, s2, fc1_w, fc1_b, fc2_w, fc2_b, fc3_w, fc3_b, x):
    return _forward_jit(w1, b1, w2, b2, s2, fc1_w, fc1_b, fc2_w, fc2_b,
                        fc3_w, fc3_b, x)
